# Initial kernel scaffold; baseline (speedup 1.0000x reference)
import jax
import jax.numpy as jnp
from jax.experimental import pallas as pl


def kernel(pos, batch, W1_0, b1_0, W1_1, b1_1, W1_2, b1_2, W2_0, b2_0, W2_1, b2_1, W2_2, b2_2, W3_0, b3_0, W3_1, b3_1, W3_2, b3_2):
    raise NotImplementedError("write your pallas kernel here")


# speedup vs baseline: 3.4152x; 3.4152x over previous
import functools

import jax
import jax.numpy as jnp
from jax import lax
from jax.experimental import pallas as pl
from jax.experimental.pallas import tpu as pltpu

B = 8
K = 64
_HI = jax.lax.Precision.HIGHEST


def _fps_kernel(m, px_ref, py_ref, pz_ref, sel_ref):
    px = px_ref[...]
    py = py_ref[...]
    pz = pz_ref[...]
    bsh = px.shape
    ssh = sel_ref.shape
    flat = (lax.broadcasted_iota(jnp.int32, bsh, 1) * 128
            + lax.broadcasted_iota(jnp.int32, bsh, 2))
    flat_sel = (lax.broadcasted_iota(jnp.int32, ssh, 1) * 128
                + lax.broadcasted_iota(jnp.int32, ssh, 2))
    mind0 = jnp.full(bsh, 1e30, dtype=jnp.float32)
    sel0 = jnp.zeros(ssh, dtype=jnp.int32)
    lx0 = px[:, 0:1, 0:1]
    ly0 = py[:, 0:1, 0:1]
    lz0 = pz[:, 0:1, 0:1]

    def body(i, st):
        sel, mind, lx, ly, lz = st
        d = (px - lx) ** 2 + (py - ly) ** 2 + (pz - lz) ** 2
        mind = jnp.minimum(mind, d)
        maxv = jnp.max(mind, axis=(1, 2), keepdims=True)
        nxt = jnp.min(
            jnp.where(mind == maxv, flat, jnp.int32(2**30)),
            axis=(1, 2), keepdims=True)
        sel = sel + jnp.where(flat_sel == i, nxt, 0)
        msk = flat == nxt
        lx = jnp.sum(jnp.where(msk, px, 0.0), axis=(1, 2), keepdims=True)
        ly = jnp.sum(jnp.where(msk, py, 0.0), axis=(1, 2), keepdims=True)
        lz = jnp.sum(jnp.where(msk, pz, 0.0), axis=(1, 2), keepdims=True)
        return sel, mind, lx, ly, lz

    sel, _, _, _, _ = lax.fori_loop(
        1, m, body, (sel0, mind0, lx0, ly0, lz0))
    sel_ref[...] = sel


def _fps_call(pos, m):
    n = pos.shape[1]
    r = n // 128
    sr = m // 128
    px = pos[:, :, 0].reshape(B, r, 128)
    py = pos[:, :, 1].reshape(B, r, 128)
    pz = pos[:, :, 2].reshape(B, r, 128)
    sel = pl.pallas_call(
        functools.partial(_fps_kernel, m),
        out_shape=jax.ShapeDtypeStruct((B, sr, 128), jnp.int32),
        interpret=False,
    )(px, py, pz)
    return sel.reshape(B, m)


def _sa_kernel(r2, f, dout, pt_ref, p_ref, sel_ref,
               w1_ref, b1_ref, w2_ref, b2_ref, w3_ref, b3_ref,
               xout_ref, cent_ref, d2_ref):
    pt = pt_ref[0]
    pm = p_ref[0]
    selv = sel_ref[0, 0]
    ch = selv.shape[0]
    n = pt.shape[1]
    ion = lax.broadcasted_iota(jnp.int32, (ch, n), 1)
    ohc = (ion == selv).astype(jnp.float32)
    centall = jnp.dot(ohc, pm, precision=_HI)
    colf = lax.broadcasted_iota(jnp.int32, (ch, f), 1)
    centpad = jnp.where(colf >= f - 3, centall, 0.0)
    c3 = centall[:, f - 3:f]
    cx = c3[:, 0:1]
    cy = c3[:, 1:2]
    cz = c3[:, 2:3]
    pxr = pt[0:1, :]
    pyr = pt[1:2, :]
    pzr = pt[2:3, :]
    d2 = (cx - pxr) ** 2 + (cy - pyr) ** 2 + (cz - pzr) ** 2
    d2_ref[...] = jnp.where(d2 <= r2, d2, 1e30)

    w1 = w1_ref[...]
    b1 = b1_ref[...]
    w2 = w2_ref[...]
    b2 = b2_ref[...]
    w3 = w3_ref[...]
    b3 = b3_ref[...]

    def body(k, out):
        dm = d2_ref[...]
        minv = jnp.min(dm, axis=1, keepdims=True)
        amin = jnp.min(
            jnp.where(dm == minv, ion, jnp.int32(2**30)),
            axis=1, keepdims=True)
        d2_ref[...] = jnp.where(ion == amin, 3e30, dm)
        oh = (ion == amin).astype(jnp.float32)
        g = jnp.dot(oh, pm, precision=_HI)
        x = g - centpad
        h = jnp.maximum(jnp.dot(x, w1) + b1, 0.0)
        h = jnp.maximum(jnp.dot(h, w2) + b2, 0.0)
        h = jnp.maximum(jnp.dot(h, w3) + b3, 0.0)
        valid = minv < 1e29
        return jnp.maximum(out, jnp.where(valid, h, -1e30))

    out0 = jnp.full((ch, dout), -1e30, dtype=jnp.float32)
    out = lax.fori_loop(0, K, body, out0)
    xout_ref[0] = out
    cent_ref[0] = c3


def _sa_call(pos, feats, sel, r, ws, bs):
    n = pos.shape[1]
    m = sel.shape[1]
    ch = 128
    mb = m // ch
    if feats is None:
        pmat = pos
    else:
        pmat = jnp.concatenate([feats, pos], axis=-1)
    f = pmat.shape[-1]
    dout = ws[2].shape[1]
    post = jnp.transpose(pos, (0, 2, 1))
    sel4 = sel.reshape(B, mb, ch, 1)
    bsr = [b.reshape(1, -1) for b in bs]
    grid = (B, mb)
    xout, cent = pl.pallas_call(
        functools.partial(_sa_kernel, r * r, f, dout),
        grid=grid,
        in_specs=[
            pl.BlockSpec((1, 3, n), lambda b, j: (b, 0, 0)),
            pl.BlockSpec((1, n, f), lambda b, j: (b, 0, 0)),
            pl.BlockSpec((1, 1, ch, 1), lambda b, j: (b, j, 0, 0)),
            pl.BlockSpec(ws[0].shape, lambda b, j: (0, 0)),
            pl.BlockSpec(bsr[0].shape, lambda b, j: (0, 0)),
            pl.BlockSpec(ws[1].shape, lambda b, j: (0, 0)),
            pl.BlockSpec(bsr[1].shape, lambda b, j: (0, 0)),
            pl.BlockSpec(ws[2].shape, lambda b, j: (0, 0)),
            pl.BlockSpec(bsr[2].shape, lambda b, j: (0, 0)),
        ],
        out_specs=[
            pl.BlockSpec((1, ch, dout), lambda b, j: (b, j, 0)),
            pl.BlockSpec((1, ch, 3), lambda b, j: (b, j, 0)),
        ],
        out_shape=[
            jax.ShapeDtypeStruct((B, m, dout), jnp.float32),
            jax.ShapeDtypeStruct((B, m, 3), jnp.float32),
        ],
        scratch_shapes=[pltpu.VMEM((ch, n), jnp.float32)],
        interpret=False,
    )(post, pmat, sel4, ws[0], bsr[0], ws[1], bsr[1], ws[2], bsr[2])
    return xout, cent


def _final_kernel(x_ref, w1_ref, b1_ref, w2_ref, b2_ref, w3_ref, b3_ref,
                  out_ref):
    x = x_ref[0]
    h = jnp.maximum(jnp.dot(x, w1_ref[...]) + b1_ref[...], 0.0)
    h = jnp.maximum(jnp.dot(h, w2_ref[...]) + b2_ref[...], 0.0)
    h = jnp.maximum(jnp.dot(h, w3_ref[...]) + b3_ref[...], 0.0)
    out_ref[0] = jnp.max(h, axis=0, keepdims=True)


def _final_call(x, ws, bs):
    m = x.shape[1]
    f = x.shape[2]
    denc = ws[2].shape[1]
    bsr = [b.reshape(1, -1) for b in bs]
    out = pl.pallas_call(
        _final_kernel,
        grid=(B,),
        in_specs=[
            pl.BlockSpec((1, m, f), lambda b: (b, 0, 0)),
            pl.BlockSpec(ws[0].shape, lambda b: (0, 0)),
            pl.BlockSpec(bsr[0].shape, lambda b: (0, 0)),
            pl.BlockSpec(ws[1].shape, lambda b: (0, 0)),
            pl.BlockSpec(bsr[1].shape, lambda b: (0, 0)),
            pl.BlockSpec(ws[2].shape, lambda b: (0, 0)),
            pl.BlockSpec(bsr[2].shape, lambda b: (0, 0)),
        ],
        out_specs=pl.BlockSpec((1, 1, denc), lambda b: (b, 0, 0)),
        out_shape=jax.ShapeDtypeStruct((B, 1, denc), jnp.float32),
        interpret=False,
    )(x, ws[0], bsr[0], ws[1], bsr[1], ws[2], bsr[2])
    return out.reshape(B, denc)


def kernel(pos, batch, W1_0, b1_0, W1_1, b1_1, W1_2, b1_2,
           W2_0, b2_0, W2_1, b2_1, W2_2, b2_2,
           W3_0, b3_0, W3_1, b3_1, W3_2, b3_2):
    pos3 = pos.reshape(B, 1024, 3)
    sel1 = _fps_call(pos3, 512)
    x1, cent1 = _sa_call(pos3, None, sel1, 0.2,
                         [W1_0, W1_1, W1_2], [b1_0, b1_1, b1_2])
    sel2 = _fps_call(cent1, 128)
    x2, cent2 = _sa_call(cent1, x1, sel2, 0.4,
                         [W2_0, W2_1, W2_2], [b2_0, b2_1, b2_2])
    xf = jnp.concatenate([x2, cent2], axis=-1)
    return _final_call(xf, [W3_0, W3_1, W3_2], [b3_0, b3_1, b3_2])


# speedup vs baseline: 4.1506x; 1.2153x over previous
import functools

import jax
import jax.numpy as jnp
from jax import lax
from jax.experimental import pallas as pl
from jax.experimental.pallas import tpu as pltpu

B = 8
K = 64
_HI = jax.lax.Precision.HIGHEST


def _fps_kernel(m, px_ref, py_ref, pz_ref, sel_ref):
    px = px_ref[...]
    py = py_ref[...]
    pz = pz_ref[...]
    bsh = px.shape
    ssh = sel_ref.shape
    flat = (lax.broadcasted_iota(jnp.int32, bsh, 1) * 128
            + lax.broadcasted_iota(jnp.int32, bsh, 2))
    flat_sel = (lax.broadcasted_iota(jnp.int32, ssh, 1) * 128
                + lax.broadcasted_iota(jnp.int32, ssh, 2))
    mind0 = jnp.full(bsh, 1e30, dtype=jnp.float32)
    sel0 = jnp.zeros(ssh, dtype=jnp.int32)
    lx0 = px[:, 0:1, 0:1]
    ly0 = py[:, 0:1, 0:1]
    lz0 = pz[:, 0:1, 0:1]

    def body(i, st):
        sel, mind, lx, ly, lz = st
        d = (px - lx) ** 2 + (py - ly) ** 2 + (pz - lz) ** 2
        mind = jnp.minimum(mind, d)
        maxv = jnp.max(mind, axis=(1, 2), keepdims=True)
        nxt = jnp.min(
            jnp.where(mind == maxv, flat, jnp.int32(2**30)),
            axis=(1, 2), keepdims=True)
        sel = sel + jnp.where(flat_sel == i, nxt, 0)
        msk = flat == nxt
        lx = jnp.sum(jnp.where(msk, px, 0.0), axis=(1, 2), keepdims=True)
        ly = jnp.sum(jnp.where(msk, py, 0.0), axis=(1, 2), keepdims=True)
        lz = jnp.sum(jnp.where(msk, pz, 0.0), axis=(1, 2), keepdims=True)
        return sel, mind, lx, ly, lz

    sel, _, _, _, _ = lax.fori_loop(
        1, m, body, (sel0, mind0, lx0, ly0, lz0))
    sel_ref[...] = sel


def _fps_call(pos, m):
    n = pos.shape[1]
    r = n // 128
    sr = m // 128
    px = pos[:, :, 0].reshape(B, r, 128)
    py = pos[:, :, 1].reshape(B, r, 128)
    pz = pos[:, :, 2].reshape(B, r, 128)
    sel = pl.pallas_call(
        functools.partial(_fps_kernel, m),
        out_shape=jax.ShapeDtypeStruct((B, sr, 128), jnp.int32),
        interpret=False,
    )(px, py, pz)
    return sel.reshape(B, m)


def _sa_kernel(r2, f, dout, pt_ref, p_ref, sel_ref,
               w1_ref, b1_ref, w2_ref, b2_ref, w3_ref, b3_ref,
               xout_ref, cent_ref, d2_ref):
    pt = pt_ref[0]
    pm = p_ref[0]
    selv = sel_ref[0, 0]
    ch = selv.shape[0]
    n = pt.shape[1]
    ion = lax.broadcasted_iota(jnp.int32, (ch, n), 1)
    ohc = (ion == selv).astype(jnp.float32)
    centall = jnp.dot(ohc, pm, precision=_HI)
    colf = lax.broadcasted_iota(jnp.int32, (ch, f), 1)
    centpad = jnp.where(colf >= f - 3, centall, 0.0)
    c3 = centall[:, f - 3:f]
    cx = c3[:, 0:1]
    cy = c3[:, 1:2]
    cz = c3[:, 2:3]
    pxr = pt[0:1, :]
    pyr = pt[1:2, :]
    pzr = pt[2:3, :]
    d2 = (cx - pxr) ** 2 + (cy - pyr) ** 2 + (cz - pzr) ** 2
    d2_ref[...] = jnp.where(d2 <= r2, d2, 1e30)

    w1 = w1_ref[...]
    b1 = b1_ref[...]
    w2 = w2_ref[...]
    b2 = b2_ref[...]
    w3 = w3_ref[...]
    b3 = b3_ref[...]

    def body(k, out):
        dm = d2_ref[...]
        minv = jnp.min(dm, axis=1, keepdims=True)
        amin = jnp.min(
            jnp.where(dm == minv, ion, jnp.int32(2**30)),
            axis=1, keepdims=True)
        d2_ref[...] = jnp.where(ion == amin, 3e30, dm)
        oh = (ion == amin).astype(jnp.float32)
        g = jnp.dot(oh, pm, precision=_HI)
        x = g - centpad
        h = jnp.maximum(jnp.dot(x, w1) + b1, 0.0)
        h = jnp.maximum(jnp.dot(h, w2) + b2, 0.0)
        h = jnp.maximum(jnp.dot(h, w3) + b3, 0.0)
        valid = minv < 1e29
        return jnp.maximum(out, jnp.where(valid, h, -1e30))

    out0 = jnp.full((ch, dout), -1e30, dtype=jnp.float32)
    out = lax.fori_loop(0, K, body, out0)
    xout_ref[0] = out
    cent_ref[0] = c3


def _sa_call(pos, feats, sel, r, ws, bs):
    n = pos.shape[1]
    m = sel.shape[1]
    ch = min(m, 256)
    mb = m // ch
    if feats is None:
        pmat = pos
    else:
        pmat = jnp.concatenate([feats, pos], axis=-1)
    f = pmat.shape[-1]
    dout = ws[2].shape[1]
    post = jnp.transpose(pos, (0, 2, 1))
    sel4 = sel.reshape(B, mb, ch, 1)
    bsr = [b.reshape(1, -1) for b in bs]
    grid = (B, mb)
    xout, cent = pl.pallas_call(
        functools.partial(_sa_kernel, r * r, f, dout),
        grid=grid,
        in_specs=[
            pl.BlockSpec((1, 3, n), lambda b, j: (b, 0, 0)),
            pl.BlockSpec((1, n, f), lambda b, j: (b, 0, 0)),
            pl.BlockSpec((1, 1, ch, 1), lambda b, j: (b, j, 0, 0)),
            pl.BlockSpec(ws[0].shape, lambda b, j: (0, 0)),
            pl.BlockSpec(bsr[0].shape, lambda b, j: (0, 0)),
            pl.BlockSpec(ws[1].shape, lambda b, j: (0, 0)),
            pl.BlockSpec(bsr[1].shape, lambda b, j: (0, 0)),
            pl.BlockSpec(ws[2].shape, lambda b, j: (0, 0)),
            pl.BlockSpec(bsr[2].shape, lambda b, j: (0, 0)),
        ],
        out_specs=[
            pl.BlockSpec((1, ch, dout), lambda b, j: (b, j, 0)),
            pl.BlockSpec((1, ch, 3), lambda b, j: (b, j, 0)),
        ],
        out_shape=[
            jax.ShapeDtypeStruct((B, m, dout), jnp.float32),
            jax.ShapeDtypeStruct((B, m, 3), jnp.float32),
        ],
        scratch_shapes=[pltpu.VMEM((ch, n), jnp.float32)],
        interpret=False,
    )(post, pmat, sel4, ws[0], bsr[0], ws[1], bsr[1], ws[2], bsr[2])
    return xout, cent


def _final_kernel(x_ref, w1_ref, b1_ref, w2_ref, b2_ref, w3_ref, b3_ref,
                  out_ref):
    x = x_ref[0]
    h = jnp.maximum(jnp.dot(x, w1_ref[...]) + b1_ref[...], 0.0)
    h = jnp.maximum(jnp.dot(h, w2_ref[...]) + b2_ref[...], 0.0)
    h = jnp.maximum(jnp.dot(h, w3_ref[...]) + b3_ref[...], 0.0)
    out_ref[0] = jnp.max(h, axis=0, keepdims=True)


def _final_call(x, ws, bs):
    m = x.shape[1]
    f = x.shape[2]
    denc = ws[2].shape[1]
    bsr = [b.reshape(1, -1) for b in bs]
    out = pl.pallas_call(
        _final_kernel,
        grid=(B,),
        in_specs=[
            pl.BlockSpec((1, m, f), lambda b: (b, 0, 0)),
            pl.BlockSpec(ws[0].shape, lambda b: (0, 0)),
            pl.BlockSpec(bsr[0].shape, lambda b: (0, 0)),
            pl.BlockSpec(ws[1].shape, lambda b: (0, 0)),
            pl.BlockSpec(bsr[1].shape, lambda b: (0, 0)),
            pl.BlockSpec(ws[2].shape, lambda b: (0, 0)),
            pl.BlockSpec(bsr[2].shape, lambda b: (0, 0)),
        ],
        out_specs=pl.BlockSpec((1, 1, denc), lambda b: (b, 0, 0)),
        out_shape=jax.ShapeDtypeStruct((B, 1, denc), jnp.float32),
        interpret=False,
    )(x, ws[0], bsr[0], ws[1], bsr[1], ws[2], bsr[2])
    return out.reshape(B, denc)


def kernel(pos, batch, W1_0, b1_0, W1_1, b1_1, W1_2, b1_2,
           W2_0, b2_0, W2_1, b2_1, W2_2, b2_2,
           W3_0, b3_0, W3_1, b3_1, W3_2, b3_2):
    pos3 = pos.reshape(B, 1024, 3)
    sel1 = _fps_call(pos3, 512)
    x1, cent1 = _sa_call(pos3, None, sel1, 0.2,
                         [W1_0, W1_1, W1_2], [b1_0, b1_1, b1_2])
    sel2 = _fps_call(cent1, 128)
    x2, cent2 = _sa_call(cent1, x1, sel2, 0.4,
                         [W2_0, W2_1, W2_2], [b2_0, b2_1, b2_2])
    xf = jnp.concatenate([x2, cent2], axis=-1)
    return _final_call(xf, [W3_0, W3_1, W3_2], [b3_0, b3_1, b3_2])


# speedup vs baseline: 4.7970x; 1.1557x over previous
import functools

import jax
import jax.numpy as jnp
from jax import lax
from jax.experimental import pallas as pl
from jax.experimental.pallas import tpu as pltpu

B = 8
K = 64
_HI = jax.lax.Precision.HIGHEST


def _fps_kernel(m, px_ref, py_ref, pz_ref, sel_ref):
    px = px_ref[...]
    py = py_ref[...]
    pz = pz_ref[...]
    bsh = px.shape
    ssh = sel_ref.shape
    flat = (lax.broadcasted_iota(jnp.int32, bsh, 1) * 128
            + lax.broadcasted_iota(jnp.int32, bsh, 2))
    flat_sel = (lax.broadcasted_iota(jnp.int32, ssh, 1) * 128
                + lax.broadcasted_iota(jnp.int32, ssh, 2))
    mind0 = jnp.full(bsh, 1e30, dtype=jnp.float32)
    sel0 = jnp.zeros(ssh, dtype=jnp.int32)
    lx0 = px[:, 0:1, 0:1]
    ly0 = py[:, 0:1, 0:1]
    lz0 = pz[:, 0:1, 0:1]

    def body(i, st):
        sel, mind, lx, ly, lz = st
        d = (px - lx) ** 2 + (py - ly) ** 2 + (pz - lz) ** 2
        mind = jnp.minimum(mind, d)
        maxv = jnp.max(mind, axis=(1, 2), keepdims=True)
        nxt = jnp.min(
            jnp.where(mind == maxv, flat, jnp.int32(2**30)),
            axis=(1, 2), keepdims=True)
        sel = sel + jnp.where(flat_sel == i, nxt, 0)
        msk = flat == nxt
        lx = jnp.sum(jnp.where(msk, px, 0.0), axis=(1, 2), keepdims=True)
        ly = jnp.sum(jnp.where(msk, py, 0.0), axis=(1, 2), keepdims=True)
        lz = jnp.sum(jnp.where(msk, pz, 0.0), axis=(1, 2), keepdims=True)
        return sel, mind, lx, ly, lz

    sel, _, _, _, _ = lax.fori_loop(
        1, m, body, (sel0, mind0, lx0, ly0, lz0))
    sel_ref[...] = sel


def _fps_call(pos, m):
    n = pos.shape[1]
    r = n // 128
    sr = m // 128
    px = pos[:, :, 0].reshape(B, r, 128)
    py = pos[:, :, 1].reshape(B, r, 128)
    pz = pos[:, :, 2].reshape(B, r, 128)
    sel = pl.pallas_call(
        functools.partial(_fps_kernel, m),
        out_shape=jax.ShapeDtypeStruct((B, sr, 128), jnp.int32),
        interpret=False,
    )(px, py, pz)
    return sel.reshape(B, m)


def _sa_kernel(r2, f, dout, pt_ref, p_ref, sel_ref,
               w1_ref, b1_ref, w2_ref, b2_ref, w3_ref, b3_ref,
               xout_ref, cent_ref, d2_ref):
    pt = pt_ref[0]
    pm = p_ref[0]
    selv = sel_ref[0, 0]
    ch = selv.shape[0]
    n = pt.shape[1]
    ion = lax.broadcasted_iota(jnp.int32, (ch, n), 1)
    ohc = (ion == selv).astype(jnp.float32)
    centall = jnp.dot(ohc, pm, precision=_HI)
    colf = lax.broadcasted_iota(jnp.int32, (ch, f), 1)
    centpad = jnp.where(colf >= f - 3, centall, 0.0)
    c3 = centall[:, f - 3:f]
    cx = c3[:, 0:1]
    cy = c3[:, 1:2]
    cz = c3[:, 2:3]
    pxr = pt[0:1, :]
    pyr = pt[1:2, :]
    pzr = pt[2:3, :]
    d2 = (cx - pxr) ** 2 + (cy - pyr) ** 2 + (cz - pzr) ** 2
    d2_ref[...] = jnp.where(d2 <= r2, d2, 1e30)

    w1 = w1_ref[...]
    b1 = b1_ref[...]
    w2 = w2_ref[...]
    b2 = b2_ref[...]
    w3 = w3_ref[...]
    b3 = b3_ref[...]

    def body(k, out):
        dm = d2_ref[...]
        minv = jnp.min(dm, axis=1, keepdims=True)
        amin = jnp.min(
            jnp.where(dm == minv, ion, jnp.int32(2**30)),
            axis=1, keepdims=True)
        d2_ref[...] = jnp.where(ion == amin, 3e30, dm)
        oh = (ion == amin).astype(jnp.float32)
        g = jnp.dot(oh, pm, precision=_HI)
        x = g - centpad
        h = jnp.maximum(jnp.dot(x, w1) + b1, 0.0)
        h = jnp.maximum(jnp.dot(h, w2) + b2, 0.0)
        h = jnp.maximum(jnp.dot(h, w3) + b3, 0.0)
        valid = minv < 1e29
        return jnp.maximum(out, jnp.where(valid, h, -1e30))

    out0 = jnp.full((ch, dout), -1e30, dtype=jnp.float32)
    out = lax.fori_loop(0, K, body, out0)
    xout_ref[0] = out
    cent_ref[0] = c3


def _sa_call(pos, feats, sel, r, ws, bs):
    n = pos.shape[1]
    m = sel.shape[1]
    ch = min(m, 512)
    mb = m // ch
    if feats is None:
        pmat = pos
    else:
        pmat = jnp.concatenate([feats, pos], axis=-1)
    f = pmat.shape[-1]
    dout = ws[2].shape[1]
    post = jnp.transpose(pos, (0, 2, 1))
    sel4 = sel.reshape(B, mb, ch, 1)
    bsr = [b.reshape(1, -1) for b in bs]
    grid = (B, mb)
    xout, cent = pl.pallas_call(
        functools.partial(_sa_kernel, r * r, f, dout),
        grid=grid,
        in_specs=[
            pl.BlockSpec((1, 3, n), lambda b, j: (b, 0, 0)),
            pl.BlockSpec((1, n, f), lambda b, j: (b, 0, 0)),
            pl.BlockSpec((1, 1, ch, 1), lambda b, j: (b, j, 0, 0)),
            pl.BlockSpec(ws[0].shape, lambda b, j: (0, 0)),
            pl.BlockSpec(bsr[0].shape, lambda b, j: (0, 0)),
            pl.BlockSpec(ws[1].shape, lambda b, j: (0, 0)),
            pl.BlockSpec(bsr[1].shape, lambda b, j: (0, 0)),
            pl.BlockSpec(ws[2].shape, lambda b, j: (0, 0)),
            pl.BlockSpec(bsr[2].shape, lambda b, j: (0, 0)),
        ],
        out_specs=[
            pl.BlockSpec((1, ch, dout), lambda b, j: (b, j, 0)),
            pl.BlockSpec((1, ch, 3), lambda b, j: (b, j, 0)),
        ],
        out_shape=[
            jax.ShapeDtypeStruct((B, m, dout), jnp.float32),
            jax.ShapeDtypeStruct((B, m, 3), jnp.float32),
        ],
        scratch_shapes=[pltpu.VMEM((ch, n), jnp.float32)],
        interpret=False,
    )(post, pmat, sel4, ws[0], bsr[0], ws[1], bsr[1], ws[2], bsr[2])
    return xout, cent


def _final_kernel(x_ref, w1_ref, b1_ref, w2_ref, b2_ref, w3_ref, b3_ref,
                  out_ref):
    x = x_ref[0]
    h = jnp.maximum(jnp.dot(x, w1_ref[...]) + b1_ref[...], 0.0)
    h = jnp.maximum(jnp.dot(h, w2_ref[...]) + b2_ref[...], 0.0)
    h = jnp.maximum(jnp.dot(h, w3_ref[...]) + b3_ref[...], 0.0)
    out_ref[0] = jnp.max(h, axis=0, keepdims=True)


def _final_call(x, ws, bs):
    m = x.shape[1]
    f = x.shape[2]
    denc = ws[2].shape[1]
    bsr = [b.reshape(1, -1) for b in bs]
    out = pl.pallas_call(
        _final_kernel,
        grid=(B,),
        in_specs=[
            pl.BlockSpec((1, m, f), lambda b: (b, 0, 0)),
            pl.BlockSpec(ws[0].shape, lambda b: (0, 0)),
            pl.BlockSpec(bsr[0].shape, lambda b: (0, 0)),
            pl.BlockSpec(ws[1].shape, lambda b: (0, 0)),
            pl.BlockSpec(bsr[1].shape, lambda b: (0, 0)),
            pl.BlockSpec(ws[2].shape, lambda b: (0, 0)),
            pl.BlockSpec(bsr[2].shape, lambda b: (0, 0)),
        ],
        out_specs=pl.BlockSpec((1, 1, denc), lambda b: (b, 0, 0)),
        out_shape=jax.ShapeDtypeStruct((B, 1, denc), jnp.float32),
        interpret=False,
    )(x, ws[0], bsr[0], ws[1], bsr[1], ws[2], bsr[2])
    return out.reshape(B, denc)


def kernel(pos, batch, W1_0, b1_0, W1_1, b1_1, W1_2, b1_2,
           W2_0, b2_0, W2_1, b2_1, W2_2, b2_2,
           W3_0, b3_0, W3_1, b3_1, W3_2, b3_2):
    pos3 = pos.reshape(B, 1024, 3)
    sel1 = _fps_call(pos3, 512)
    x1, cent1 = _sa_call(pos3, None, sel1, 0.2,
                         [W1_0, W1_1, W1_2], [b1_0, b1_1, b1_2])
    sel2 = _fps_call(cent1, 128)
    x2, cent2 = _sa_call(cent1, x1, sel2, 0.4,
                         [W2_0, W2_1, W2_2], [b2_0, b2_1, b2_2])
    xf = jnp.concatenate([x2, cent2], axis=-1)
    return _final_call(xf, [W3_0, W3_1, W3_2], [b3_0, b3_1, b3_2])
